# Initial kernel scaffold; baseline (speedup 1.0000x reference)
import jax
import jax.numpy as jnp
from jax.experimental import pallas as pl


def kernel(tags, tag_embedding_weight):
    raise NotImplementedError("write your pallas kernel here")


# speedup vs baseline: 6.5991x; 6.5991x over previous
import functools

import jax
import jax.numpy as jnp
from jax import lax
from jax.experimental import pallas as pl
from jax.experimental.pallas import tpu as pltpu
from jax.experimental.pallas import tpu_sc as plsc

NC = 2
NS = 16
NW = NC * NS
CHUNK = 128


def _body(nchunk, table_hbm, idx_hbm, out_hbm, idx_v, rows_v, gsem):
    wid = lax.axis_index("s") * NC + lax.axis_index("c")
    pltpu.sync_copy(idx_hbm.at[wid], idx_v)
    pltpu.async_copy(table_hbm.at[idx_v.at[0]], rows_v.at[0], gsem)

    @pl.loop(0, nchunk, step=2)
    def _chunks(g):
        for b in range(2):
            j = g + b
            pltpu.make_async_copy(
                table_hbm.at[idx_v.at[j]], rows_v.at[b], gsem
            ).wait()

            @pl.when(j + 1 < nchunk)
            def _start_next():
                pltpu.async_copy(
                    table_hbm.at[idx_v.at[j + 1]], rows_v.at[1 - b], gsem
                )

            pltpu.sync_copy(rows_v.at[b], out_hbm.at[wid, j])


def kernel(tags, tag_embedding_weight):
    B_total = tags.shape[0] * tags.shape[1]
    D = tag_embedding_weight.shape[1]
    assert B_total % (NW * CHUNK) == 0
    nchunk = B_total // (NW * CHUNK)
    assert nchunk % 2 == 0

    idx = tags.reshape(NW, nchunk, CHUNK).astype(jnp.int32)

    mesh = plsc.VectorSubcoreMesh(
        core_axis_name="c", subcore_axis_name="s", num_cores=NC, num_subcores=NS
    )
    out = pl.kernel(
        functools.partial(_body, nchunk),
        out_type=jax.ShapeDtypeStruct((NW, nchunk, CHUNK, D), jnp.float32),
        mesh=mesh,
        scratch_types=[
            pltpu.VMEM((nchunk, CHUNK), jnp.int32),
            pltpu.VMEM((2, CHUNK, D), jnp.float32),
            pltpu.SemaphoreType.DMA,
        ],
    )(tag_embedding_weight, idx)
    return out.reshape(tags.shape[0], tags.shape[1], D)


# speedup vs baseline: 7.6963x; 1.1663x over previous
import functools

import jax
import jax.numpy as jnp
from jax import lax
from jax.experimental import pallas as pl
from jax.experimental.pallas import tpu as pltpu
from jax.experimental.pallas import tpu_sc as plsc

NC = 2
NS = 16
NW = NC * NS
CHUNK = 128
GC = 2
NSLOT = 3


def _body(nchunk, table_hbm, idx_hbm, out_hbm, idx_v, rows_v, gsem, w0, w1, w2):
    wsem = (w0, w1, w2)
    wid = lax.axis_index("s") * NC + lax.axis_index("c")
    ngroups = nchunk // GC
    pltpu.sync_copy(idx_hbm.at[wid], idx_v)

    def start_gathers(grp, slot):
        for c in range(GC):
            pltpu.async_copy(
                table_hbm.at[idx_v.at[grp * GC + c]], rows_v.at[slot, c], gsem
            )

    def drain_gathers(grp, slot):
        for c in range(GC):
            pltpu.make_async_copy(
                table_hbm.at[idx_v.at[grp * GC + c]], rows_v.at[slot, c], gsem
            ).wait()

    def write_descriptor(grp, slot):
        return pltpu.make_async_copy(
            rows_v.at[slot], out_hbm.at[wid, pl.ds(grp * GC, GC)], wsem[slot]
        )

    start_gathers(0, 0)

    @pl.loop(0, ngroups, step=NSLOT)
    def _groups(g0):
        for slot in range(NSLOT):
            grp = g0 + slot

            @pl.when(grp < ngroups)
            def _do_group():
                drain_gathers(grp, slot)
                nslot = (slot + 1) % NSLOT

                @pl.when(grp + 1 < ngroups)
                def _prefetch_next():
                    @pl.when(grp >= 2)
                    def _free_slot():
                        write_descriptor(grp - 2, nslot).wait()

                    start_gathers(grp + 1, nslot)

                write_descriptor(grp, slot).start()

    for grp in (ngroups - 2, ngroups - 1):
        write_descriptor(grp, grp % NSLOT).wait()


def kernel(tags, tag_embedding_weight):
    B_total = tags.shape[0] * tags.shape[1]
    D = tag_embedding_weight.shape[1]
    assert B_total % (NW * CHUNK) == 0
    nchunk = B_total // (NW * CHUNK)
    assert nchunk % GC == 0 and nchunk // GC >= NSLOT

    idx = tags.reshape(NW, nchunk, CHUNK).astype(jnp.int32)

    mesh = plsc.VectorSubcoreMesh(
        core_axis_name="c", subcore_axis_name="s", num_cores=NC, num_subcores=NS
    )
    out = pl.kernel(
        functools.partial(_body, nchunk),
        out_type=jax.ShapeDtypeStruct((NW, nchunk, CHUNK, D), jnp.float32),
        mesh=mesh,
        scratch_types=[
            pltpu.VMEM((nchunk, CHUNK), jnp.int32),
            pltpu.VMEM((NSLOT, GC, CHUNK, D), jnp.float32),
            pltpu.SemaphoreType.DMA,
            pltpu.SemaphoreType.DMA,
            pltpu.SemaphoreType.DMA,
            pltpu.SemaphoreType.DMA,
        ],
    )(tag_embedding_weight, idx)
    return out.reshape(tags.shape[0], tags.shape[1], D)


# speedup vs baseline: 7.8751x; 1.0232x over previous
import functools

import jax
import jax.numpy as jnp
from jax import lax
from jax.experimental import pallas as pl
from jax.experimental.pallas import tpu as pltpu
from jax.experimental.pallas import tpu_sc as plsc

NC = 2
NS = 16
NW = NC * NS
CHUNK = 128
GC = 2
NSLOT = 3


def _body(nchunk, table_hbm, idx_hbm, out_hbm, idx_v, rows_v, *sems):
    gsem, wsem = sems[:NSLOT], sems[NSLOT:]
    wid = lax.axis_index("s") * NC + lax.axis_index("c")
    ngroups = nchunk // GC
    pltpu.sync_copy(idx_hbm.at[wid], idx_v)

    def gather_descriptor(grp, chunk, slot):
        return pltpu.make_async_copy(
            table_hbm.at[idx_v.at[grp * GC + chunk]],
            rows_v.at[slot, chunk],
            gsem[slot],
        )

    def write_descriptor(grp, slot):
        return pltpu.make_async_copy(
            rows_v.at[slot], out_hbm.at[wid, pl.ds(grp * GC, GC)], wsem[slot]
        )

    def start_gathers(grp, slot):
        for c in range(GC):
            gather_descriptor(grp, c, slot).start()

    start_gathers(0, 0)

    @pl.loop(0, ngroups, step=NSLOT)
    def _groups(g0):
        for slot in range(NSLOT):
            grp = g0 + slot

            @pl.when(grp < ngroups)
            def _do_group():
                nslot = (slot + 1) % NSLOT

                @pl.when(grp + 1 < ngroups)
                def _prefetch_next():
                    @pl.when(grp >= 2)
                    def _free_slot():
                        write_descriptor(grp - 2, nslot).wait()

                    start_gathers(grp + 1, nslot)

                for c in range(GC):
                    gather_descriptor(grp, c, slot).wait()
                write_descriptor(grp, slot).start()

    for grp in (ngroups - 2, ngroups - 1):
        write_descriptor(grp, grp % NSLOT).wait()


def kernel(tags, tag_embedding_weight):
    B_total = tags.shape[0] * tags.shape[1]
    D = tag_embedding_weight.shape[1]
    assert B_total % (NW * CHUNK) == 0
    nchunk = B_total // (NW * CHUNK)
    assert nchunk % GC == 0 and nchunk // GC >= NSLOT

    idx = tags.reshape(NW, nchunk, CHUNK).astype(jnp.int32)

    mesh = plsc.VectorSubcoreMesh(
        core_axis_name="c", subcore_axis_name="s", num_cores=NC, num_subcores=NS
    )
    out = pl.kernel(
        functools.partial(_body, nchunk),
        out_type=jax.ShapeDtypeStruct((NW, nchunk, CHUNK, D), jnp.float32),
        mesh=mesh,
        scratch_types=[
            pltpu.VMEM((nchunk, CHUNK), jnp.int32),
            pltpu.VMEM((NSLOT, GC, CHUNK, D), jnp.float32),
        ] + [pltpu.SemaphoreType.DMA] * (2 * NSLOT),
    )(tag_embedding_weight, idx)
    return out.reshape(tags.shape[0], tags.shape[1], D)


# speedup vs baseline: 8.0152x; 1.0178x over previous
import functools

import jax
import jax.numpy as jnp
from jax import lax
from jax.experimental import pallas as pl
from jax.experimental.pallas import tpu as pltpu
from jax.experimental.pallas import tpu_sc as plsc

NC = 2
NS = 16
NW = NC * NS
CHUNK = 128
GC = 2
NSLOT = 3


def _body(nchunk, table_hbm, idx_hbm, out_hbm, idx_v, rows_v, *sems):
    gsem, wsem = sems[:NSLOT], sems[NSLOT:]
    wid = lax.axis_index("s") * NC + lax.axis_index("c")
    ngroups = nchunk // GC
    pltpu.sync_copy(idx_hbm.at[wid], idx_v)

    def gather_descriptor(grp, chunk, slot):
        return pltpu.make_async_copy(
            table_hbm.at[idx_v.at[grp * GC + chunk]],
            rows_v.at[slot, chunk],
            gsem[slot],
        )

    def write_descriptor(grp, chunk, slot):
        return pltpu.make_async_copy(
            rows_v.at[slot, chunk], out_hbm.at[wid, grp * GC + chunk], wsem[slot]
        )

    def start_gathers(grp, slot):
        for c in range(GC):
            gather_descriptor(grp, c, slot).start()

    start_gathers(0, 0)

    @pl.loop(0, ngroups, step=NSLOT)
    def _groups(g0):
        for slot in range(NSLOT):
            grp = g0 + slot

            @pl.when(grp < ngroups)
            def _do_group():
                nslot = (slot + 1) % NSLOT

                @pl.when(grp + 1 < ngroups)
                def _prefetch_next():
                    @pl.when(grp >= 2)
                    def _free_slot():
                        for c in range(GC):
                            write_descriptor(grp - 2, c, nslot).wait()

                    start_gathers(grp + 1, nslot)

                for c in range(GC):
                    gather_descriptor(grp, c, slot).wait()
                    write_descriptor(grp, c, slot).start()

    for grp in (ngroups - 2, ngroups - 1):
        for c in range(GC):
            write_descriptor(grp, c, grp % NSLOT).wait()


def kernel(tags, tag_embedding_weight):
    B_total = tags.shape[0] * tags.shape[1]
    D = tag_embedding_weight.shape[1]
    assert B_total % (NW * CHUNK) == 0
    nchunk = B_total // (NW * CHUNK)
    assert nchunk % GC == 0 and nchunk // GC >= NSLOT

    idx = tags.reshape(NW, nchunk, CHUNK).astype(jnp.int32)

    mesh = plsc.VectorSubcoreMesh(
        core_axis_name="c", subcore_axis_name="s", num_cores=NC, num_subcores=NS
    )
    out = pl.kernel(
        functools.partial(_body, nchunk),
        out_type=jax.ShapeDtypeStruct((NW, nchunk, CHUNK, D), jnp.float32),
        mesh=mesh,
        scratch_types=[
            pltpu.VMEM((nchunk, CHUNK), jnp.int32),
            pltpu.VMEM((NSLOT, GC, CHUNK, D), jnp.float32),
        ] + [pltpu.SemaphoreType.DMA] * (2 * NSLOT),
    )(tag_embedding_weight, idx)
    return out.reshape(tags.shape[0], tags.shape[1], D)
